# baseline (speedup 1.0000x reference)
import functools
import math

import jax
import jax.numpy as jnp
from jax import lax
from jax.experimental import pallas as pl
from jax.experimental.pallas import tpu as pltpu
from jax.experimental.pallas import tpu_sc as plsc

NUM_BUCKETS = 32
H = 16
Q = 2048
K = 2048
NWORKERS = 32
ROWS_PER_W = Q // NWORKERS
NGROUPS = K // 16


def _inner_bucket_lut():
    d = jnp.arange(-K, K, dtype=jnp.int32)
    nb = NUM_BUCKETS // 2
    max_exact = nb // 2
    relb = (d > 0).astype(jnp.int32) * nb
    a = jnp.abs(d)
    is_small = a < max_exact
    rel_if_large = max_exact + (
        jnp.log(jnp.maximum(a, 1).astype(jnp.float32) / max_exact)
        / math.log(128 / max_exact)
        * (nb - max_exact)
    ).astype(jnp.int32)
    rel_if_large = jnp.minimum(rel_if_large, nb - 1)
    return relb + jnp.where(is_small, a, rel_if_large)


def _sc_body(rb_hbm, lut_hbm, tab_hbm, out_hbm,
             rb_v, lut_v, tab_v, out_v,
             sem_in0, sem_in1, sem_out0, sem_out1):
    wid = lax.axis_index("s") * 2 + lax.axis_index("c")
    q0 = wid * ROWS_PER_W

    pltpu.sync_copy(lut_hbm, lut_v)
    pltpu.sync_copy(tab_hbm, tab_v)

    sems_in = (sem_in0, sem_in1)
    sems_out = (sem_out0, sem_out1)
    kiota = lax.iota(jnp.int32, 16)

    pltpu.async_copy(rb_hbm.at[q0], rb_v.at[0], sem_in0)
    pltpu.async_copy(rb_hbm.at[q0 + 1], rb_v.at[1], sem_in1)

    def row_pair(i, carry):
        for b in range(2):
            r = 2 * i + b
            q = q0 + r

            pltpu.make_async_copy(rb_hbm.at[q], rb_v.at[b], sems_in[b]).wait()
            @pl.when(i > 0)
            def _wait_out():
                pltpu.make_async_copy(
                    out_v.at[b], out_hbm.at[:, q, :], sems_out[b]).wait()

            @plsc.parallel_loop(0, NGROUPS, 1, unroll=4)
            def group(j):
                rb = rb_v[b, pl.ds(j * 16, 16)]
                lidx = kiota + (j * 16 + K - q)
                inner = plsc.load_gather(lut_v, [lidx])
                base = jnp.where(rb == 0, inner, rb + 31)
                for h0 in range(0, H, 8):
                    vals = [plsc.load_gather(tab_v, [base + (h0 + h) * 64])
                            for h in range(8)]
                    for h in range(8):
                        out_v[b, h0 + h, pl.ds(j * 16, 16)] = vals[h]

            pltpu.async_copy(out_v.at[b], out_hbm.at[:, q, :], sems_out[b])

            @pl.when(i < ROWS_PER_W // 2 - 1)
            def _prefetch():
                pltpu.async_copy(rb_hbm.at[q + 2], rb_v.at[b], sems_in[b])
        return carry

    lax.fori_loop(0, ROWS_PER_W // 2, row_pair, 0, unroll=False)

    qlast = q0 + ROWS_PER_W - 2
    pltpu.make_async_copy(out_v.at[0], out_hbm.at[:, qlast, :], sem_out0).wait()
    pltpu.make_async_copy(out_v.at[1], out_hbm.at[:, qlast + 1, :], sem_out1).wait()


@jax.jit
def _bias_sc(rel_buckets2d, lut, tab_flat):
    mesh = plsc.VectorSubcoreMesh(
        core_axis_name="c", subcore_axis_name="s", num_cores=2,
        num_subcores=16)
    run = functools.partial(
        pl.kernel,
        out_type=jax.ShapeDtypeStruct((H, Q, K), jnp.float32),
        mesh=mesh,
        scratch_types=[
            pltpu.VMEM((2, K), jnp.int32),
            pltpu.VMEM((2 * K,), jnp.int32),
            pltpu.VMEM((64 * H,), jnp.float32),
            pltpu.VMEM((2, H, K), jnp.float32),
            pltpu.SemaphoreType.DMA,
            pltpu.SemaphoreType.DMA,
            pltpu.SemaphoreType.DMA,
            pltpu.SemaphoreType.DMA,
        ],
        compiler_params=pltpu.CompilerParams(needs_layout_passes=False),
    )(_sc_body)
    return run(rel_buckets2d, lut, tab_flat)


def kernel(query_pos, key_pos, rel_buckets, rel_bias):
    lut = _inner_bucket_lut()
    out = _bias_sc(rel_buckets.reshape(Q, K), lut, rel_bias.T.reshape(-1))
    return out.reshape(1, H, Q, K)


# speedup vs baseline: 1.1957x; 1.1957x over previous
import functools
import math

import jax
import jax.numpy as jnp
from jax import lax
from jax.experimental import pallas as pl
from jax.experimental.pallas import tpu as pltpu
from jax.experimental.pallas import tpu_sc as plsc

NUM_BUCKETS = 32
H = 16
Q = 2048
K = 2048
NWORKERS = 32
ROWS_PER_W = Q // NWORKERS
NGROUPS = K // 16


def _inner_bucket_lut():
    d = jnp.arange(-K, K, dtype=jnp.int32)
    nb = NUM_BUCKETS // 2
    max_exact = nb // 2
    relb = (d > 0).astype(jnp.int32) * nb
    a = jnp.abs(d)
    is_small = a < max_exact
    rel_if_large = max_exact + (
        jnp.log(jnp.maximum(a, 1).astype(jnp.float32) / max_exact)
        / math.log(128 / max_exact)
        * (nb - max_exact)
    ).astype(jnp.int32)
    rel_if_large = jnp.minimum(rel_if_large, nb - 1)
    return relb + jnp.where(is_small, a, rel_if_large)


def _sc_body(rb_hbm, lut_hbm, tab_hbm, out_hbm,
             rb_v, lut_v, tab_v, out_v,
             sem_in0, sem_in1, sem_out0, sem_out1):
    wid = lax.axis_index("s") * 2 + lax.axis_index("c")
    q0 = wid * ROWS_PER_W

    pltpu.sync_copy(lut_hbm, lut_v)
    pltpu.sync_copy(tab_hbm, tab_v)

    sems_in = (sem_in0, sem_in1)
    sems_out = (sem_out0, sem_out1)
    kiota = lax.iota(jnp.int32, 16)

    pltpu.async_copy(rb_hbm.at[q0], rb_v.at[0], sem_in0)
    pltpu.async_copy(rb_hbm.at[q0 + 1], rb_v.at[1], sem_in1)

    def row_pair(i, carry):
        for b in range(2):
            r = 2 * i + b
            q = q0 + r

            pltpu.make_async_copy(rb_hbm.at[q], rb_v.at[b], sems_in[b]).wait()
            @pl.when(i > 0)
            def _wait_out():
                pltpu.make_async_copy(
                    out_v.at[b], out_hbm.at[:, q, :], sems_out[b]).wait()

            @plsc.parallel_loop(0, NGROUPS, 1, unroll=2)
            def group(j):
                rb = rb_v[b, pl.ds(j * 16, 16)]
                lidx = kiota + (j * 16 + K - q)
                inner = plsc.load_gather(lut_v, [lidx])
                base = jnp.where(rb == 0, inner, rb + 31)
                for h0 in range(0, H, 4):
                    vals = [plsc.load_gather(tab_v, [base + (h0 + h) * 64])
                            for h in range(4)]
                    for h in range(4):
                        out_v[b, h0 + h, pl.ds(j * 16, 16)] = vals[h]

            pltpu.async_copy(out_v.at[b], out_hbm.at[:, q, :], sems_out[b])

            @pl.when(i < ROWS_PER_W // 2 - 1)
            def _prefetch():
                pltpu.async_copy(rb_hbm.at[q + 2], rb_v.at[b], sems_in[b])
        return carry

    lax.fori_loop(0, ROWS_PER_W // 2, row_pair, 0, unroll=False)

    qlast = q0 + ROWS_PER_W - 2
    pltpu.make_async_copy(out_v.at[0], out_hbm.at[:, qlast, :], sem_out0).wait()
    pltpu.make_async_copy(out_v.at[1], out_hbm.at[:, qlast + 1, :], sem_out1).wait()


@jax.jit
def _bias_sc(rel_buckets2d, lut, tab_flat):
    mesh = plsc.VectorSubcoreMesh(
        core_axis_name="c", subcore_axis_name="s", num_cores=2,
        num_subcores=16)
    run = functools.partial(
        pl.kernel,
        out_type=jax.ShapeDtypeStruct((H, Q, K), jnp.float32),
        mesh=mesh,
        scratch_types=[
            pltpu.VMEM((2, K), jnp.int32),
            pltpu.VMEM((2 * K,), jnp.int32),
            pltpu.VMEM((64 * H,), jnp.float32),
            pltpu.VMEM((2, H, K), jnp.float32),
            pltpu.SemaphoreType.DMA,
            pltpu.SemaphoreType.DMA,
            pltpu.SemaphoreType.DMA,
            pltpu.SemaphoreType.DMA,
        ],
        compiler_params=pltpu.CompilerParams(needs_layout_passes=False),
    )(_sc_body)
    return run(rel_buckets2d, lut, tab_flat)


def kernel(query_pos, key_pos, rel_buckets, rel_bias):
    lut = _inner_bucket_lut()
    out = _bias_sc(rel_buckets.reshape(Q, K), lut, rel_bias.T.reshape(-1))
    return out.reshape(1, H, Q, K)


# speedup vs baseline: 1.2201x; 1.0204x over previous
import functools
import math

import jax
import jax.numpy as jnp
from jax import lax
from jax.experimental import pallas as pl
from jax.experimental.pallas import tpu as pltpu
from jax.experimental.pallas import tpu_sc as plsc

NUM_BUCKETS = 32
H = 16
Q = 2048
K = 2048
NWORKERS = 32
ROWS_PER_W = Q // NWORKERS
NGROUPS = K // 16


def _inner_bucket_lut():
    d = jnp.arange(-K, K, dtype=jnp.int32)
    nb = NUM_BUCKETS // 2
    max_exact = nb // 2
    relb = (d > 0).astype(jnp.int32) * nb
    a = jnp.abs(d)
    is_small = a < max_exact
    rel_if_large = max_exact + (
        jnp.log(jnp.maximum(a, 1).astype(jnp.float32) / max_exact)
        / math.log(128 / max_exact)
        * (nb - max_exact)
    ).astype(jnp.int32)
    rel_if_large = jnp.minimum(rel_if_large, nb - 1)
    return relb + jnp.where(is_small, a, rel_if_large)


def _sc_body(rb_hbm, lut_hbm, tab_hbm, out_hbm,
             rb_v, lut_v, tab_v, out_v,
             sem_in0, sem_in1, sem_out0, sem_out1):
    wid = lax.axis_index("s") * 2 + lax.axis_index("c")
    q0 = wid * ROWS_PER_W

    pltpu.sync_copy(lut_hbm, lut_v)
    pltpu.sync_copy(tab_hbm, tab_v)

    sems_in = (sem_in0, sem_in1)
    sems_out = (sem_out0, sem_out1)
    kiota = lax.iota(jnp.int32, 16)

    pltpu.async_copy(rb_hbm.at[q0], rb_v.at[0], sem_in0)
    pltpu.async_copy(rb_hbm.at[q0 + 1], rb_v.at[1], sem_in1)

    def row_pair(i, carry):
        for b in range(2):
            r = 2 * i + b
            q = q0 + r

            pltpu.make_async_copy(rb_hbm.at[q], rb_v.at[b], sems_in[b]).wait()
            @pl.when(i > 0)
            def _wait_out():
                pltpu.make_async_copy(
                    out_v.at[b], out_hbm.at[:, q, :], sems_out[b]).wait()

            @plsc.parallel_loop(0, NGROUPS, 1, unroll=2)
            def group(j):
                rb = rb_v[b, pl.ds(j * 16, 16)]
                lidx = kiota + (j * 16 + K - q)
                inner = plsc.load_gather(lut_v, [lidx])
                base = jnp.where(rb == 0, inner, rb + 31)
                for h0 in range(0, H, 2):
                    vals = [plsc.load_gather(tab_v, [base + (h0 + h) * 64])
                            for h in range(2)]
                    for h in range(2):
                        out_v[b, h0 + h, pl.ds(j * 16, 16)] = vals[h]

            pltpu.async_copy(out_v.at[b], out_hbm.at[:, q, :], sems_out[b])

            @pl.when(i < ROWS_PER_W // 2 - 1)
            def _prefetch():
                pltpu.async_copy(rb_hbm.at[q + 2], rb_v.at[b], sems_in[b])
        return carry

    lax.fori_loop(0, ROWS_PER_W // 2, row_pair, 0, unroll=False)

    qlast = q0 + ROWS_PER_W - 2
    pltpu.make_async_copy(out_v.at[0], out_hbm.at[:, qlast, :], sem_out0).wait()
    pltpu.make_async_copy(out_v.at[1], out_hbm.at[:, qlast + 1, :], sem_out1).wait()


@jax.jit
def _bias_sc(rel_buckets2d, lut, tab_flat):
    mesh = plsc.VectorSubcoreMesh(
        core_axis_name="c", subcore_axis_name="s", num_cores=2,
        num_subcores=16)
    run = functools.partial(
        pl.kernel,
        out_type=jax.ShapeDtypeStruct((H, Q, K), jnp.float32),
        mesh=mesh,
        scratch_types=[
            pltpu.VMEM((2, K), jnp.int32),
            pltpu.VMEM((2 * K,), jnp.int32),
            pltpu.VMEM((64 * H,), jnp.float32),
            pltpu.VMEM((2, H, K), jnp.float32),
            pltpu.SemaphoreType.DMA,
            pltpu.SemaphoreType.DMA,
            pltpu.SemaphoreType.DMA,
            pltpu.SemaphoreType.DMA,
        ],
        compiler_params=pltpu.CompilerParams(needs_layout_passes=False),
    )(_sc_body)
    return run(rel_buckets2d, lut, tab_flat)


def kernel(query_pos, key_pos, rel_buckets, rel_bias):
    lut = _inner_bucket_lut()
    out = _bias_sc(rel_buckets.reshape(Q, K), lut, rel_bias.T.reshape(-1))
    return out.reshape(1, H, Q, K)


# speedup vs baseline: 1.2495x; 1.0241x over previous
import functools
import math

import jax
import jax.numpy as jnp
from jax import lax
from jax.experimental import pallas as pl
from jax.experimental.pallas import tpu as pltpu
from jax.experimental.pallas import tpu_sc as plsc

NUM_BUCKETS = 32
H = 16
Q = 2048
K = 2048
NWORKERS = 32
ROWS_PER_W = Q // NWORKERS
NGROUPS = K // 16


def _inner_bucket_lut():
    d = jnp.arange(-K, K, dtype=jnp.int32)
    nb = NUM_BUCKETS // 2
    max_exact = nb // 2
    relb = (d > 0).astype(jnp.int32) * nb
    a = jnp.abs(d)
    is_small = a < max_exact
    rel_if_large = max_exact + (
        jnp.log(jnp.maximum(a, 1).astype(jnp.float32) / max_exact)
        / math.log(128 / max_exact)
        * (nb - max_exact)
    ).astype(jnp.int32)
    rel_if_large = jnp.minimum(rel_if_large, nb - 1)
    return relb + jnp.where(is_small, a, rel_if_large)


def _sc_body(rb_hbm, lut_hbm, tab_hbm, out_hbm,
             rb_v, lut_v, tab_v, out_v,
             sem_in0, sem_in1, sem_out0, sem_out1):
    wid = lax.axis_index("s") * 2 + lax.axis_index("c")
    q0 = wid * ROWS_PER_W

    pltpu.sync_copy(lut_hbm, lut_v)
    pltpu.sync_copy(tab_hbm, tab_v)

    sems_in = (sem_in0, sem_in1)
    sems_out = (sem_out0, sem_out1)
    kiota = lax.iota(jnp.int32, 16)

    pltpu.async_copy(rb_hbm.at[q0], rb_v.at[0], sem_in0)
    pltpu.async_copy(rb_hbm.at[q0 + 1], rb_v.at[1], sem_in1)

    def row_pair(i, carry):
        for b in range(2):
            r = 2 * i + b
            q = q0 + r

            pltpu.make_async_copy(rb_hbm.at[q], rb_v.at[b], sems_in[b]).wait()
            @pl.when(i > 0)
            def _wait_out():
                pltpu.make_async_copy(
                    out_v.at[b], out_hbm.at[:, q, :], sems_out[b]).wait()

            @plsc.parallel_loop(0, NGROUPS, 1, unroll=2)
            def group(j):
                rb = rb_v[b, pl.ds(j * 16, 16)]
                lidx = kiota + (j * 16 + K - q)
                inner = plsc.load_gather(lut_v, [lidx])
                base = jnp.where(rb == 0, inner, rb + 31)
                for h in range(H):
                    out_v[b, h, pl.ds(j * 16, 16)] = plsc.load_gather(
                        tab_v, [base + h * 64])

            pltpu.async_copy(out_v.at[b], out_hbm.at[:, q, :], sems_out[b])

            @pl.when(i < ROWS_PER_W // 2 - 1)
            def _prefetch():
                pltpu.async_copy(rb_hbm.at[q + 2], rb_v.at[b], sems_in[b])
        return carry

    lax.fori_loop(0, ROWS_PER_W // 2, row_pair, 0, unroll=False)

    qlast = q0 + ROWS_PER_W - 2
    pltpu.make_async_copy(out_v.at[0], out_hbm.at[:, qlast, :], sem_out0).wait()
    pltpu.make_async_copy(out_v.at[1], out_hbm.at[:, qlast + 1, :], sem_out1).wait()


@jax.jit
def _bias_sc(rel_buckets2d, lut, tab_flat):
    mesh = plsc.VectorSubcoreMesh(
        core_axis_name="c", subcore_axis_name="s", num_cores=2,
        num_subcores=16)
    run = functools.partial(
        pl.kernel,
        out_type=jax.ShapeDtypeStruct((H, Q, K), jnp.float32),
        mesh=mesh,
        scratch_types=[
            pltpu.VMEM((2, K), jnp.int32),
            pltpu.VMEM((2 * K,), jnp.int32),
            pltpu.VMEM((64 * H,), jnp.float32),
            pltpu.VMEM((2, H, K), jnp.float32),
            pltpu.SemaphoreType.DMA,
            pltpu.SemaphoreType.DMA,
            pltpu.SemaphoreType.DMA,
            pltpu.SemaphoreType.DMA,
        ],
        compiler_params=pltpu.CompilerParams(needs_layout_passes=False),
    )(_sc_body)
    return run(rel_buckets2d, lut, tab_flat)


def kernel(query_pos, key_pos, rel_buckets, rel_bias):
    lut = _inner_bucket_lut()
    out = _bias_sc(rel_buckets.reshape(Q, K), lut, rel_bias.T.reshape(-1))
    return out.reshape(1, H, Q, K)
